# baseline (speedup 1.0000x reference)
import functools

import jax
import jax.numpy as jnp
from jax import lax
from jax.experimental import pallas as pl
from jax.experimental.pallas import tpu as pltpu
from jax.experimental.pallas import tpu_sc as plsc

B = 32
L = 50
NE = 34
NA = 36
D = 128
H = 256
NG = NA - 1 + NE - 1
_F32 = jnp.float32


def _sc_gather(table, idx):
    info = plsc.get_sparse_core_info()
    nw = info.num_cores * info.num_subcores
    n = idx.shape[0]
    per = n // nw

    mesh = plsc.VectorSubcoreMesh(core_axis_name="c", subcore_axis_name="s")

    @functools.partial(
        pl.kernel,
        mesh=mesh,
        out_type=jax.ShapeDtypeStruct((n, D), _F32),
        scratch_types=[
            pltpu.VMEM((per,), jnp.int32),
            pltpu.VMEM((per, D), _F32),
            pltpu.SemaphoreType.DMA,
        ],
    )
    def gather_kernel(table_hbm, idx_hbm, out_hbm, idx_v, rows_v, sem):
        wid = lax.axis_index("s") * info.num_cores + lax.axis_index("c")
        base = wid * per
        pltpu.sync_copy(idx_hbm.at[pl.ds(base, per)], idx_v)
        pltpu.async_copy(table_hbm.at[idx_v], rows_v, sem).wait()
        pltpu.sync_copy(rows_v, out_hbm.at[pl.ds(base, per)])

    return gather_kernel(table, idx)


def _fused_body(emb_ref, wifb_ref, whf_ref, bf_ref, whb_ref, bb_ref,
                wpf_ref, wpb_ref,
                weg_ref, bev_ref, wg_ref, ba_ref,
                ev_out_ref, ar_out_ref,
                xf_ref, xb_ref, hf_ref, hb_ref,
                e1_ref, a2f_ref, a2b_ref, a1b_ref, gtrg_ref, gcat_ref):
    i = pl.program_id(0)

    @pl.when(i == 0)
    def _pre():
        emb = emb_ref[...].reshape(L * B, D)
        xfb = jnp.dot(emb, wifb_ref[...], preferred_element_type=_F32)
        xf_ref[...] = xfb[:, 0:4 * H].reshape(L, B, 4 * H)
        xb_ref[...] = xfb[:, 4 * H:8 * H].reshape(L, B, 4 * H)
        whf = whf_ref[...]
        whb = whb_ref[...]
        bf = bf_ref[...]
        bb = bb_ref[...]
        zero = jnp.zeros((B, H), _F32)

        def cell(gates, c):
            ig = jax.nn.sigmoid(gates[:, 0:H])
            f = jax.nn.sigmoid(gates[:, H:2 * H])
            g = jnp.tanh(gates[:, 2 * H:3 * H])
            o = jax.nn.sigmoid(gates[:, 3 * H:4 * H])
            c = f * c + ig * g
            return o * jnp.tanh(c), c

        def step(k, carry):
            hfv, cf, hbv, cb = carry
            tb = L - 1 - k
            gf = (xf_ref[k]
                  + jnp.dot(hfv, whf, preferred_element_type=_F32)) + bf
            gb = (xb_ref[tb]
                  + jnp.dot(hbv, whb, preferred_element_type=_F32)) + bb
            hfv, cf = cell(gf, cf)
            hbv, cb = cell(gb, cb)
            hf_ref[k] = hfv
            hb_ref[tb] = hbv
            return (hfv, cf, hbv, cb)

        lax.fori_loop(0, L, step, (zero, zero, zero, zero))

        hf = hf_ref[...].reshape(L * B, H)
        hb = hb_ref[...].reshape(L * B, H)
        pf = jnp.dot(hf, wpf_ref[...], preferred_element_type=_F32)
        pb = jnp.dot(hb, wpb_ref[...], preferred_element_type=_F32)
        e1_ref[...] = (pf[:, 0:NE] + pb[:, 0:NE]).reshape(L, B, NE)
        a2f_ref[...] = pf[:, NE:NE + NA].reshape(L, B, NA)
        a2b_ref[...] = pb[:, NE:NE + NA].reshape(L, B, NA)
        a1 = (pf[:, NE + NA:NE + 2 * NA]
              + pb[:, NE + NA:NE + 2 * NA]).reshape(L, B, NA)
        for b in range(B):
            a1b_ref[pl.ds(b * L, L), :] = a1[:, b, :]
        gtrg_ref[...] = jnp.zeros((B, NE - 1), _F32)
        gcat_ref[...] = jnp.zeros((B * L, NG), _F32)

    @pl.when(i > 0)
    def _step():
        t = i - 1
        g_trg = gtrg_ref[...]
        g_cat = gcat_ref[...]

        ev = (e1_ref[t]
              + jnp.dot(g_trg, weg_ref[...], preferred_element_type=_F32)
              ) + bev_ref[...]
        ev_out_ref[:, 0, 0, :] = ev
        iota_ne = lax.broadcasted_iota(jnp.int32, (B, NE), 1)
        ev_pred = jnp.min(
            jnp.where(ev == jnp.max(ev, axis=1, keepdims=True), iota_ne, NE),
            axis=1, keepdims=True)

        a1b3 = a1b_ref[...].reshape(B, L, NA)
        gdot3 = jnp.dot(
            g_cat, wg_ref[...], preferred_element_type=_F32
        ).reshape(B, L, NA)
        ar3 = (((a1b3 + a2f_ref[t][:, None, :]) + a2b_ref[t][:, None, :])
               + gdot3) + ba_ref[...][None]
        ar_out_ref[:, 0, :, :] = ar3
        iota3_na = lax.broadcasted_iota(jnp.int32, (B, L, NA), 2)
        a_pred = jnp.min(
            jnp.where(ar3 == jnp.max(ar3, axis=2, keepdims=True),
                      iota3_na, NA),
            axis=2, keepdims=True)

        ev_mask = ev_pred > 0
        e_idx = jnp.maximum(ev_pred - 1, 0)
        arg_mask = a_pred > 0
        a_idx = jnp.maximum(a_pred - 1, 0)
        iota_e = lax.broadcasted_iota(jnp.int32, (B, NE - 1), 1)
        iota3_g = lax.broadcasted_iota(jnp.int32, (B, L, NG), 2)
        gtrg_ref[...] = jnp.maximum(
            g_trg, jnp.where((iota_e == e_idx) & ev_mask, 1.0, 0.0))
        gcat_ref[...] = jnp.maximum(
            g_cat,
            jnp.where(((iota3_g == a_idx) | (iota3_g == e_idx[:, None, :]
                                             + (NA - 1)))
                      & arg_mask & ev_mask[:, None, :], 1.0, 0.0
                      ).reshape(B * L, NG))


def _run_fused(emb, wifb, whf_t, bf, whb_t, bb,
               wpf, wpb, w_event, b_event, w_arg, b_arg):
    weg = w_event[:, 2 * H:2 * H + NE - 1].T
    wg = w_arg[:, 4 * H:4 * H + NG].T
    def whole(x):
        nd = len(x.shape)
        return pl.BlockSpec(x.shape, lambda i, _n=nd: (0,) * _n)

    shift = lambda i: (0, jnp.maximum(i - 1, 0), 0, 0)
    ins = [emb, wifb, whf_t, bf, whb_t, bb, wpf, wpb,
           weg, b_event.reshape(1, NE), wg, b_arg.reshape(1, NA)]
    return pl.pallas_call(
        _fused_body,
        grid=(L + 1,),
        in_specs=[whole(x) for x in ins],
        out_specs=[
            pl.BlockSpec((B, 1, 1, NE), shift),
            pl.BlockSpec((B, 1, L, NA), shift),
        ],
        out_shape=[
            jax.ShapeDtypeStruct((B, L, 1, NE), _F32),
            jax.ShapeDtypeStruct((B, L, L, NA), _F32),
        ],
        scratch_shapes=[
            pltpu.VMEM((L, B, 4 * H), _F32),
            pltpu.VMEM((L, B, 4 * H), _F32),
            pltpu.VMEM((L, B, H), _F32),
            pltpu.VMEM((L, B, H), _F32),
            pltpu.VMEM((L, B, NE), _F32),
            pltpu.VMEM((L, B, NA), _F32),
            pltpu.VMEM((L, B, NA), _F32),
            pltpu.VMEM((B * L, NA), _F32),
            pltpu.VMEM((B, NE - 1), _F32),
            pltpu.VMEM((B * L, NG), _F32),
        ],
    )(*ins)


def kernel(input_ids, embedding, W_ih_f, W_hh_f, b_f, W_ih_b, W_hh_b, b_b,
           W_event, b_event, W_arg, b_arg):
    idx = input_ids.astype(jnp.int32).T.reshape(L * B)
    n_pad = 64 * 32
    idx_pad = jnp.concatenate(
        [idx, jnp.zeros((n_pad - L * B,), jnp.int32)])
    rows = _sc_gather(embedding.astype(_F32), idx_pad)
    emb = rows[:L * B].reshape(L, B, D)

    wifb = jnp.concatenate([W_ih_f.T, W_ih_b.T], axis=1)
    wpf = jnp.concatenate(
        [W_event[:, 0:H].T, W_arg[:, 2 * H:3 * H].T, W_arg[:, 0:H].T], axis=1)
    wpb = jnp.concatenate(
        [W_event[:, H:2 * H].T, W_arg[:, 3 * H:4 * H].T,
         W_arg[:, H:2 * H].T], axis=1)
    ev, ar = _run_fused(
        emb, wifb, W_hh_f.T, b_f.reshape(1, 4 * H),
        W_hh_b.T, b_b.reshape(1, 4 * H),
        wpf, wpb, W_event, b_event, W_arg, b_arg)
    return ev.reshape(B, L, NE), ar


# speedup vs baseline: 1.3715x; 1.3715x over previous
import functools

import jax
import jax.numpy as jnp
from jax import lax
from jax.experimental import pallas as pl
from jax.experimental.pallas import tpu as pltpu
from jax.experimental.pallas import tpu_sc as plsc

B = 32
L = 50
NE = 34
NA = 36
D = 128
H = 256
NG = NA - 1 + NE - 1
_F32 = jnp.float32


def _sc_gather(table, idx):
    info = plsc.get_sparse_core_info()
    nw = info.num_cores * info.num_subcores
    n = idx.shape[0]
    per = n // nw

    mesh = plsc.VectorSubcoreMesh(core_axis_name="c", subcore_axis_name="s")

    @functools.partial(
        pl.kernel,
        mesh=mesh,
        out_type=jax.ShapeDtypeStruct((n, D), _F32),
        scratch_types=[
            pltpu.VMEM((per,), jnp.int32),
            pltpu.VMEM((per, D), _F32),
            pltpu.SemaphoreType.DMA,
        ],
    )
    def gather_kernel(table_hbm, idx_hbm, out_hbm, idx_v, rows_v, sem):
        wid = lax.axis_index("s") * info.num_cores + lax.axis_index("c")
        base = wid * per
        pltpu.sync_copy(idx_hbm.at[pl.ds(base, per)], idx_v)
        pltpu.async_copy(table_hbm.at[idx_v], rows_v, sem).wait()
        pltpu.sync_copy(rows_v, out_hbm.at[pl.ds(base, per)])

    return gather_kernel(table, idx)


def _fused_body(emb_ref, wifb_ref, whf_ref, bf_ref, whb_ref, bb_ref,
                wpf_ref, wpb_ref,
                weg_ref, bev_ref, wg_ref, ba_ref,
                ev_out_ref, ar_out_ref,
                xf_ref, xb_ref, hf_ref, hb_ref,
                e1_ref, a2f_ref, a2b_ref, a1b_ref, gtrg_ref, gcat_ref):
    i = pl.program_id(0)

    @pl.when(i == 0)
    def _pre():
        emb = emb_ref[...].reshape(L * B, D)
        xfb = jnp.dot(emb, wifb_ref[...], preferred_element_type=_F32)
        xf_ref[...] = xfb[:, 0:4 * H].reshape(L, B, 4 * H)
        xb_ref[...] = xfb[:, 4 * H:8 * H].reshape(L, B, 4 * H)
        whf = whf_ref[...]
        whb = whb_ref[...]
        bf = bf_ref[...]
        bb = bb_ref[...]
        zero = jnp.zeros((B, H), _F32)

        def cell(gates, c):
            ig = jax.nn.sigmoid(gates[:, 0:H])
            f = jax.nn.sigmoid(gates[:, H:2 * H])
            g = jnp.tanh(gates[:, 2 * H:3 * H])
            o = jax.nn.sigmoid(gates[:, 3 * H:4 * H])
            c = f * c + ig * g
            return o * jnp.tanh(c), c

        def step(k, carry):
            hfv, cf, hbv, cb = carry
            tb = L - 1 - k
            gf = (xf_ref[k]
                  + jnp.dot(hfv, whf, preferred_element_type=_F32)) + bf
            gb = (xb_ref[tb]
                  + jnp.dot(hbv, whb, preferred_element_type=_F32)) + bb
            hfv, cf = cell(gf, cf)
            hbv, cb = cell(gb, cb)
            hf_ref[k] = hfv
            hb_ref[tb] = hbv
            return (hfv, cf, hbv, cb)

        lax.fori_loop(0, L, step, (zero, zero, zero, zero))

        hf = hf_ref[...].reshape(L * B, H)
        hb = hb_ref[...].reshape(L * B, H)
        pf = jnp.dot(hf, wpf_ref[...], preferred_element_type=_F32)
        pb = jnp.dot(hb, wpb_ref[...], preferred_element_type=_F32)
        e1_ref[...] = (pf[:, 0:NE] + pb[:, 0:NE]).reshape(L, B, NE)
        a2f_ref[...] = pf[:, NE:NE + NA].reshape(L, B, NA)
        a2b_ref[...] = pb[:, NE:NE + NA].reshape(L, B, NA)
        a1 = (pf[:, NE + NA:NE + 2 * NA]
              + pb[:, NE + NA:NE + 2 * NA]).reshape(L, B, NA)
        for b in range(B):
            a1b_ref[pl.ds(b * L, L), :] = a1[:, b, :]
        gtrg_ref[...] = jnp.zeros((B, NE - 1), _F32)
        gcat_ref[...] = jnp.zeros((B * L, NG), _F32)

    @pl.when(i > 0)
    def _step():
        t = i - 1
        g_trg = gtrg_ref[...]
        g_cat = gcat_ref[...]

        ev = (e1_ref[t]
              + jnp.dot(g_trg, weg_ref[...], preferred_element_type=_F32)
              ) + bev_ref[...]
        ev_out_ref[:, 0, 0, :] = ev
        iota_ne = lax.broadcasted_iota(jnp.int32, (B, NE), 1)
        ev_pred = jnp.min(
            jnp.where(ev == jnp.max(ev, axis=1, keepdims=True), iota_ne, NE),
            axis=1, keepdims=True)

        def bc(x):
            return jnp.broadcast_to(
                x[:, None, :], (B, L, NA)).reshape(B * L, NA)

        ar = (((a1b_ref[...] + bc(a2f_ref[t])) + bc(a2b_ref[t]))
              + jnp.dot(g_cat, wg_ref[...], preferred_element_type=_F32)
              ) + ba_ref[...]
        ar_out_ref[:, 0, :, :] = ar.reshape(B, L, NA)
        iota_na = lax.broadcasted_iota(jnp.int32, (B * L, NA), 1)
        a_pred = jnp.min(
            jnp.where(ar == jnp.max(ar, axis=1, keepdims=True), iota_na, NA),
            axis=1, keepdims=True)

        ev_mask = ev_pred > 0
        e_idx = jnp.maximum(ev_pred - 1, 0)
        arg_mask = a_pred > 0
        a_idx = jnp.maximum(a_pred - 1, 0)
        evm_f = jnp.broadcast_to(
            ev_mask[:, None, :], (B, L, 1)).reshape(B * L, 1)
        eix_f = jnp.broadcast_to(
            e_idx[:, None, :], (B, L, 1)).reshape(B * L, 1)
        iota_e = lax.broadcasted_iota(jnp.int32, (B, NE - 1), 1)
        iota_g = lax.broadcasted_iota(jnp.int32, (B * L, NG), 1)
        gtrg_ref[...] = jnp.maximum(
            g_trg, jnp.where((iota_e == e_idx) & ev_mask, 1.0, 0.0))
        gcat_ref[...] = jnp.maximum(
            g_cat,
            jnp.where(((iota_g == a_idx) | (iota_g == eix_f + (NA - 1)))
                      & arg_mask & evm_f, 1.0, 0.0))


def _run_fused(emb, wifb, whf_t, bf, whb_t, bb,
               wpf, wpb, w_event, b_event, w_arg, b_arg):
    weg = w_event[:, 2 * H:2 * H + NE - 1].T
    wg = w_arg[:, 4 * H:4 * H + NG].T
    def whole(x):
        nd = len(x.shape)
        return pl.BlockSpec(x.shape, lambda i, _n=nd: (0,) * _n)

    shift = lambda i: (0, jnp.maximum(i - 1, 0), 0, 0)
    ins = [emb, wifb, whf_t, bf, whb_t, bb, wpf, wpb,
           weg, b_event.reshape(1, NE), wg, b_arg.reshape(1, NA)]
    return pl.pallas_call(
        _fused_body,
        grid=(L + 1,),
        in_specs=[whole(x) for x in ins],
        out_specs=[
            pl.BlockSpec((B, 1, 1, NE), shift),
            pl.BlockSpec((B, 1, L, NA), shift),
        ],
        out_shape=[
            jax.ShapeDtypeStruct((B, L, 1, NE), _F32),
            jax.ShapeDtypeStruct((B, L, L, NA), _F32),
        ],
        scratch_shapes=[
            pltpu.VMEM((L, B, 4 * H), _F32),
            pltpu.VMEM((L, B, 4 * H), _F32),
            pltpu.VMEM((L, B, H), _F32),
            pltpu.VMEM((L, B, H), _F32),
            pltpu.VMEM((L, B, NE), _F32),
            pltpu.VMEM((L, B, NA), _F32),
            pltpu.VMEM((L, B, NA), _F32),
            pltpu.VMEM((B * L, NA), _F32),
            pltpu.VMEM((B, NE - 1), _F32),
            pltpu.VMEM((B * L, NG), _F32),
        ],
    )(*ins)


def kernel(input_ids, embedding, W_ih_f, W_hh_f, b_f, W_ih_b, W_hh_b, b_b,
           W_event, b_event, W_arg, b_arg):
    idx = input_ids.astype(jnp.int32).T.reshape(L * B)
    n_pad = 64 * 32
    idx_pad = jnp.concatenate(
        [idx, jnp.zeros((n_pad - L * B,), jnp.int32)])
    rows = _sc_gather(embedding.astype(_F32), idx_pad)
    emb = rows[:L * B].reshape(L, B, D)

    wifb = jnp.concatenate([W_ih_f.T, W_ih_b.T], axis=1)
    wpf = jnp.concatenate(
        [W_event[:, 0:H].T, W_arg[:, 2 * H:3 * H].T, W_arg[:, 0:H].T], axis=1)
    wpb = jnp.concatenate(
        [W_event[:, H:2 * H].T, W_arg[:, 3 * H:4 * H].T,
         W_arg[:, H:2 * H].T], axis=1)
    ev, ar = _run_fused(
        emb, wifb, W_hh_f.T, b_f.reshape(1, 4 * H),
        W_hh_b.T, b_b.reshape(1, 4 * H),
        wpf, wpb, W_event, b_event, W_arg, b_arg)
    return ev.reshape(B, L, NE), ar
